# baseline (speedup 1.0000x reference)
import jax
import jax.numpy as jnp
from jax.experimental import pallas as pl
from jax.experimental.pallas import tpu as pltpu

_LATENT = 512
_W_PRED, _W_EPIST, _W_NOVEL = 1.0, 0.5, 0.5
_EPS = 1e-5


def _body(z_t_ref, act_ref, z_t1_ref, sigma_ref,
          w1a_ref, w1b_ref, b1_ref, g1_ref, bt1_ref,
          w2_ref, b2_ref, g2_ref, bt2_ref,
          w3_ref, b3_ref,
          total_ref, pred_ref, epi_ref, nov_ref):
    x1 = jnp.dot(z_t_ref[...], w1a_ref[...],
                 preferred_element_type=jnp.float32)
    x1 = x1 + jnp.dot(act_ref[...], w1b_ref[...],
                      preferred_element_type=jnp.float32)
    x1 = x1 + b1_ref[...]
    mu1 = jnp.mean(x1, axis=-1, keepdims=True)
    var1 = jnp.mean((x1 - mu1) ** 2, axis=-1, keepdims=True)
    h1 = (x1 - mu1) * jax.lax.rsqrt(var1 + _EPS) * g1_ref[...] + bt1_ref[...]
    h1 = jnp.maximum(h1, 0.0)

    x2 = jnp.dot(h1, w2_ref[...],
                 preferred_element_type=jnp.float32) + b2_ref[...]
    mu2 = jnp.mean(x2, axis=-1, keepdims=True)
    var2 = jnp.mean((x2 - mu2) ** 2, axis=-1, keepdims=True)
    h2 = (x2 - mu2) * jax.lax.rsqrt(var2 + _EPS) * g2_ref[...] + bt2_ref[...]
    h2 = jnp.maximum(h2, 0.0)

    z_pred = jnp.dot(h2, w3_ref[...],
                     preferred_element_type=jnp.float32) + b3_ref[...]
    d = z_pred - z_t1_ref[...]
    e = d * d
    pred = jax.lax.dot_general(
        jnp.ones((1, _LATENT), jnp.float32), e,
        (((1,), (1,)), ((), ())),
        preferred_element_type=jnp.float32) * (1.0 / _LATENT)

    epi = jnp.sum(sigma_ref[...], axis=0, keepdims=True) * (1.0 / 6.0)

    pred_ref[...] = pred
    epi_ref[...] = epi
    nov_ref[...] = jnp.ones_like(pred)
    total_ref[...] = _W_PRED * pred + _W_EPIST * epi + _W_NOVEL


def kernel(z_t, action, z_t1, sigma, W1, b1, g1, bt1, W2, b2, g2, bt2, W3, b3):
    B = z_t.shape[0]
    BS = 4096
    grid = B // BS

    w1a = W1[:_LATENT]
    w1b = jnp.zeros((8, 128), jnp.float32).at[:6].set(W1[_LATENT:])
    w2c = W2
    w3c = W3
    act_pad = jnp.zeros((B, 8), jnp.float32).at[:, :6].set(action)
    sig_t = jnp.zeros((8, B), jnp.float32).at[:6].set(sigma.T)

    b1r = b1.reshape(1, -1)
    g1r = g1.reshape(1, -1)
    bt1r = bt1.reshape(1, -1)
    b2r = b2.reshape(1, -1)
    g2r = g2.reshape(1, -1)
    bt2r = bt2.reshape(1, -1)
    b3r = b3.reshape(1, -1)

    def row_spec(width):
        return pl.BlockSpec((BS, width), lambda i: (i, 0))

    def const_spec(shape):
        return pl.BlockSpec(shape, lambda i: tuple(0 for _ in shape))

    out_spec = pl.BlockSpec((1, BS), lambda i: (0, i))
    out_sds = jax.ShapeDtypeStruct((1, B), jnp.float32)

    total, pred, epi, nov = pl.pallas_call(
        _body,
        grid=(grid,),
        in_specs=[
            row_spec(_LATENT),
            row_spec(8),
            row_spec(_LATENT),
            pl.BlockSpec((8, BS), lambda i: (0, i)),

            const_spec((_LATENT, 128)),
            const_spec((8, 128)),
            const_spec((1, 128)),
            const_spec((1, 128)),
            const_spec((1, 128)),
            const_spec((128, 64)),
            const_spec((1, 64)),
            const_spec((1, 64)),
            const_spec((1, 64)),
            const_spec((64, _LATENT)),
            const_spec((1, _LATENT)),
        ],
        out_specs=[out_spec, out_spec, out_spec, out_spec],
        out_shape=[out_sds, out_sds, out_sds, out_sds],
        compiler_params=pltpu.CompilerParams(
            dimension_semantics=("parallel",),
        ),
    )(z_t, act_pad, z_t1, sig_t,
      w1a, w1b, b1r, g1r, bt1r,
      w2c, b2r, g2r, bt2r, w3c, b3r)

    return (total.reshape(B), pred.reshape(B), epi.reshape(B), nov.reshape(B))


# speedup vs baseline: 1.0052x; 1.0052x over previous
import jax
import jax.numpy as jnp
from jax.experimental import pallas as pl
from jax.experimental.pallas import tpu as pltpu

_LATENT = 512
_W_PRED, _W_EPIST, _W_NOVEL = 1.0, 0.5, 0.5
_EPS = 1e-5


def _body(z_t_ref, act_ref, z_t1_ref, sigma_ref,
          w1a_ref, w1b_ref, b1_ref, g1_ref, bt1_ref,
          w2_ref, b2_ref, g2_ref, bt2_ref,
          w3_ref, b3_ref,
          total_ref, pred_ref, epi_ref, nov_ref):
    x1 = jnp.dot(z_t_ref[...], w1a_ref[...],
                 preferred_element_type=jnp.float32)
    x1 = x1 + jnp.dot(act_ref[...], w1b_ref[...],
                      preferred_element_type=jnp.float32)
    x1 = x1 + b1_ref[...]
    mu1 = jnp.mean(x1, axis=-1, keepdims=True)
    var1 = jnp.mean((x1 - mu1) ** 2, axis=-1, keepdims=True)
    h1 = (x1 - mu1) * jax.lax.rsqrt(var1 + _EPS) * g1_ref[...] + bt1_ref[...]
    h1 = jnp.maximum(h1, 0.0)

    x2 = jnp.dot(h1, w2_ref[...],
                 preferred_element_type=jnp.float32) + b2_ref[...]
    mu2 = jnp.mean(x2, axis=-1, keepdims=True)
    var2 = jnp.mean((x2 - mu2) ** 2, axis=-1, keepdims=True)
    h2 = (x2 - mu2) * jax.lax.rsqrt(var2 + _EPS) * g2_ref[...] + bt2_ref[...]
    h2 = jnp.maximum(h2, 0.0)

    z_pred = jnp.dot(h2, w3_ref[...],
                     preferred_element_type=jnp.float32) + b3_ref[...]
    d = z_pred - z_t1_ref[...]
    e = d * d
    pred = jax.lax.dot_general(
        jnp.ones((1, _LATENT), jnp.float32), e,
        (((1,), (1,)), ((), ())),
        preferred_element_type=jnp.float32) * (1.0 / _LATENT)

    epi = jnp.sum(sigma_ref[...], axis=0, keepdims=True) * (1.0 / 6.0)

    pred_ref[...] = pred
    epi_ref[...] = epi
    nov_ref[...] = jnp.ones_like(pred)
    total_ref[...] = _W_PRED * pred + _W_EPIST * epi + _W_NOVEL


def kernel(z_t, action, z_t1, sigma, W1, b1, g1, bt1, W2, b2, g2, bt2, W3, b3):
    B = z_t.shape[0]
    BS = 2048
    grid = B // BS

    w1a = W1[:_LATENT]
    w1b = jnp.zeros((8, 128), jnp.float32).at[:6].set(W1[_LATENT:])
    w2c = W2
    w3c = W3
    act_pad = jnp.zeros((B, 8), jnp.float32).at[:, :6].set(action)
    sig_t = jnp.zeros((8, B), jnp.float32).at[:6].set(sigma.T)

    b1r = b1.reshape(1, -1)
    g1r = g1.reshape(1, -1)
    bt1r = bt1.reshape(1, -1)
    b2r = b2.reshape(1, -1)
    g2r = g2.reshape(1, -1)
    bt2r = bt2.reshape(1, -1)
    b3r = b3.reshape(1, -1)

    def row_spec(width):
        return pl.BlockSpec((BS, width), lambda i: (i, 0))

    def const_spec(shape):
        return pl.BlockSpec(shape, lambda i: tuple(0 for _ in shape))

    out_spec = pl.BlockSpec((1, BS), lambda i: (0, i))
    out_sds = jax.ShapeDtypeStruct((1, B), jnp.float32)

    total, pred, epi, nov = pl.pallas_call(
        _body,
        grid=(grid,),
        in_specs=[
            row_spec(_LATENT),
            row_spec(8),
            row_spec(_LATENT),
            pl.BlockSpec((8, BS), lambda i: (0, i)),

            const_spec((_LATENT, 128)),
            const_spec((8, 128)),
            const_spec((1, 128)),
            const_spec((1, 128)),
            const_spec((1, 128)),
            const_spec((128, 64)),
            const_spec((1, 64)),
            const_spec((1, 64)),
            const_spec((1, 64)),
            const_spec((64, _LATENT)),
            const_spec((1, _LATENT)),
        ],
        out_specs=[out_spec, out_spec, out_spec, out_spec],
        out_shape=[out_sds, out_sds, out_sds, out_sds],
        compiler_params=pltpu.CompilerParams(
            dimension_semantics=("parallel",),
        ),
    )(z_t, act_pad, z_t1, sig_t,
      w1a, w1b, b1r, g1r, bt1r,
      w2c, b2r, g2r, bt2r, w3c, b3r)

    return (total.reshape(B), pred.reshape(B), epi.reshape(B), nov.reshape(B))


# speedup vs baseline: 1.0514x; 1.0460x over previous
import jax
import jax.numpy as jnp
from jax.experimental import pallas as pl
from jax.experimental.pallas import tpu as pltpu

_LATENT = 512
_W_PRED, _W_EPIST, _W_NOVEL = 1.0, 0.5, 0.5
_EPS = 1e-5


def _body(z_t_ref, act_ref, z_t1_ref, sigma_ref,
          w1_ref, b1_ref, g1_ref, bt1_ref,
          w2_ref, b2_ref, g2_ref, bt2_ref,
          w3_ref, b3_ref,
          total_ref, pred_ref, epi_ref, nov_ref):
    w1a = w1_ref[:_LATENT, :]
    w1b = w1_ref[_LATENT:, :]
    x1 = jnp.dot(z_t_ref[...], w1a, preferred_element_type=jnp.float32)
    x1 = x1 + jnp.dot(act_ref[...], w1b, preferred_element_type=jnp.float32)
    x1 = x1 + b1_ref[...]
    mu1 = jnp.mean(x1, axis=-1, keepdims=True)
    var1 = jnp.mean((x1 - mu1) ** 2, axis=-1, keepdims=True)
    h1 = (x1 - mu1) * jax.lax.rsqrt(var1 + _EPS) * g1_ref[...] + bt1_ref[...]
    h1 = jnp.maximum(h1, 0.0)

    x2 = jnp.dot(h1, w2_ref[...], preferred_element_type=jnp.float32) + b2_ref[...]
    mu2 = jnp.mean(x2, axis=-1, keepdims=True)
    var2 = jnp.mean((x2 - mu2) ** 2, axis=-1, keepdims=True)
    h2 = (x2 - mu2) * jax.lax.rsqrt(var2 + _EPS) * g2_ref[...] + bt2_ref[...]
    h2 = jnp.maximum(h2, 0.0)

    z_pred = jnp.dot(h2, w3_ref[...], preferred_element_type=jnp.float32) + b3_ref[...]
    d = z_pred - z_t1_ref[...]
    e = d * d
    pred = jax.lax.dot_general(
        jnp.ones((1, _LATENT), jnp.float32), e,
        (((1,), (1,)), ((), ())),
        preferred_element_type=jnp.float32) * (1.0 / _LATENT)

    epi = jax.lax.dot_general(
        jnp.ones((1, 6), jnp.float32), sigma_ref[...],
        (((1,), (1,)), ((), ())),
        preferred_element_type=jnp.float32) * (1.0 / 6.0)

    pred_ref[...] = pred
    epi_ref[...] = epi
    nov_ref[...] = jnp.ones_like(pred)
    total_ref[...] = _W_PRED * pred + _W_EPIST * epi + _W_NOVEL


def kernel(z_t, action, z_t1, sigma, W1, b1, g1, bt1, W2, b2, g2, bt2, W3, b3):
    B = z_t.shape[0]
    BS = 2048
    grid = B // BS

    def row_spec(width):
        return pl.BlockSpec((BS, width), lambda i: (i, 0))

    def const_spec(shape):
        return pl.BlockSpec(shape, lambda i: tuple(0 for _ in shape))

    out_spec = pl.BlockSpec((1, BS), lambda i: (0, i))
    out_sds = jax.ShapeDtypeStruct((1, B), jnp.float32)

    total, pred, epi, nov = pl.pallas_call(
        _body,
        grid=(grid,),
        in_specs=[
            row_spec(_LATENT),
            row_spec(6),
            row_spec(_LATENT),
            row_spec(6),
            const_spec((_LATENT + 6, 128)),
            const_spec((1, 128)),
            const_spec((1, 128)),
            const_spec((1, 128)),
            const_spec((128, 64)),
            const_spec((1, 64)),
            const_spec((1, 64)),
            const_spec((1, 64)),
            const_spec((64, _LATENT)),
            const_spec((1, _LATENT)),
        ],
        out_specs=[out_spec, out_spec, out_spec, out_spec],
        out_shape=[out_sds, out_sds, out_sds, out_sds],
        compiler_params=pltpu.CompilerParams(
            dimension_semantics=("parallel",),
        ),
    )(z_t, action, z_t1, sigma,
      W1, b1.reshape(1, -1), g1.reshape(1, -1), bt1.reshape(1, -1),
      W2, b2.reshape(1, -1), g2.reshape(1, -1), bt2.reshape(1, -1),
      W3, b3.reshape(1, -1))

    return (total.reshape(B), pred.reshape(B), epi.reshape(B), nov.reshape(B))
